# Initial kernel scaffold; baseline (speedup 1.0000x reference)
import jax
import jax.numpy as jnp
from jax.experimental import pallas as pl


def kernel(memory, veh_idx, veh_repr, cust_repr, edge_emb, W_in, b_in, W_h, b_h):
    raise NotImplementedError("write your pallas kernel here")


# speedup vs baseline: 209.8369x; 209.8369x over previous
import functools

import jax
import jax.numpy as jnp
from jax.experimental import pallas as pl
from jax.experimental.pallas import tpu as pltpu

N, L, H, D = 16384, 20, 128, 128
BN = 256


def _body(vi_ref, veh_ref, cust_ref, edge_ref, win_ref, bias_ref, wh_ref,
          mem_ref, out_ref):
    mem = mem_ref[...]
    vi = vi_ref[...]
    slot = jax.lax.broadcasted_iota(jnp.int32, (BN, L, 1), 1)
    mask = slot == vi[:, :, None]
    cur_h = jnp.sum(jnp.where(mask, mem, 0.0), axis=1)
    pre = jnp.dot(veh_ref[...], win_ref[0:D, :],
                  preferred_element_type=jnp.float32)
    pre += jnp.dot(cust_ref[...], win_ref[D:2 * D, :],
                   preferred_element_type=jnp.float32)
    pre += jnp.dot(edge_ref[...], win_ref[2 * D:3 * D, :],
                   preferred_element_type=jnp.float32)
    pre += jnp.dot(cur_h, wh_ref[...], preferred_element_type=jnp.float32)
    next_h = jnp.tanh(pre + bias_ref[...])
    out_ref[...] = jnp.where(mask, next_h[:, None, :], mem)


@jax.jit
def kernel(memory, veh_idx, veh_repr, cust_repr, edge_emb, W_in, b_in,
           W_h, b_h):
    n, l, h = memory.shape
    grid = n // BN
    bias = (b_in + b_h).reshape(1, h)
    row = lambda i: (i, 0)
    row3 = lambda i: (i, 0, 0)
    full = lambda i: (0, 0)
    out = pl.pallas_call(
        _body,
        grid=(grid,),
        in_specs=[
            pl.BlockSpec((BN, 1), row),
            pl.BlockSpec((BN, D), row),
            pl.BlockSpec((BN, D), row),
            pl.BlockSpec((BN, D), row),
            pl.BlockSpec((3 * D, h), full),
            pl.BlockSpec((1, h), full),
            pl.BlockSpec((D, h), full),
            pl.BlockSpec((BN, l, h), row3),
        ],
        out_specs=pl.BlockSpec((BN, l, h), row3),
        out_shape=jax.ShapeDtypeStruct((n, l, h), memory.dtype),
    )(veh_idx, veh_repr[:, 0, :], cust_repr[:, 0, :], edge_emb[:, 0, 0, :],
      W_in, bias, W_h, memory)
    return out


# speedup vs baseline: 216.2979x; 1.0308x over previous
import functools

import jax
import jax.numpy as jnp
from jax.experimental import pallas as pl
from jax.experimental.pallas import tpu as pltpu

N, L, H, D = 16384, 20, 128, 128
BN = 512


def _body(vi_ref, veh_ref, cust_ref, edge_ref, win_ref, bias_ref, wh_ref,
          mem_ref, out_ref):
    mem = mem_ref[...]
    vi = vi_ref[...]
    slot = jax.lax.broadcasted_iota(jnp.int32, (BN, L, 1), 1)
    mask = slot == vi[:, :, None]
    cur_h = jnp.sum(jnp.where(mask, mem, 0.0), axis=1)
    pre = jnp.dot(veh_ref[...], win_ref[0:D, :],
                  preferred_element_type=jnp.float32)
    pre += jnp.dot(cust_ref[...], win_ref[D:2 * D, :],
                   preferred_element_type=jnp.float32)
    pre += jnp.dot(edge_ref[...], win_ref[2 * D:3 * D, :],
                   preferred_element_type=jnp.float32)
    pre += jnp.dot(cur_h, wh_ref[...], preferred_element_type=jnp.float32)
    next_h = jnp.tanh(pre + bias_ref[...])
    out_ref[...] = jnp.where(mask, next_h[:, None, :], mem)


@jax.jit
def kernel(memory, veh_idx, veh_repr, cust_repr, edge_emb, W_in, b_in,
           W_h, b_h):
    n, l, h = memory.shape
    grid = n // BN
    bias = (b_in + b_h).reshape(1, h)
    row = lambda i: (i, 0)
    row3 = lambda i: (i, 0, 0)
    full = lambda i: (0, 0)
    out = pl.pallas_call(
        _body,
        grid=(grid,),
        in_specs=[
            pl.BlockSpec((BN, 1), row),
            pl.BlockSpec((BN, D), row),
            pl.BlockSpec((BN, D), row),
            pl.BlockSpec((BN, D), row),
            pl.BlockSpec((3 * D, h), full),
            pl.BlockSpec((1, h), full),
            pl.BlockSpec((D, h), full),
            pl.BlockSpec((BN, l, h), row3),
        ],
        out_specs=pl.BlockSpec((BN, l, h), row3),
        out_shape=jax.ShapeDtypeStruct((n, l, h), memory.dtype),
    )(veh_idx, veh_repr[:, 0, :], cust_repr[:, 0, :], edge_emb[:, 0, 0, :],
      W_in, bias, W_h, memory)
    return out
